# Initial kernel scaffold; baseline (speedup 1.0000x reference)
import jax
import jax.numpy as jnp
from jax.experimental import pallas as pl


def kernel(query, reference_points, input_flatten, input_spatial_shapes, input_level_start_index, W_so, b_so, W_aw, b_aw, W_v, b_v, W_o, b_o):
    raise NotImplementedError("write your pallas kernel here")


# speedup vs baseline: 58.9156x; 58.9156x over previous
import functools

import jax
import jax.numpy as jnp
from jax import lax
from jax.experimental import pallas as pl
from jax.experimental.pallas import tpu as pltpu
from jax.experimental.pallas import tpu_sc as plsc

D = 256
NH = 8
NL = 4
NP = 4
HD = D // NH
SPATIAL = ((64, 64), (32, 32), (16, 16), (8, 8))
STARTS = (0, 4096, 5120, 5376)
LEN_IN = 5440
B = 2
LQ = 5440
ROWS = B * LQ
SBLK = 640
GRID = ROWS // SBLK
NWORK = 32
RW = ROWS // NWORK
CH = 4
LANES = NH * NL * NP


def _proj_kernel(q_ref, inp_ref, refx_ref, refy_ref,
                 wx_ref, wy_ref, waw_ref, wv_ref,
                 bx_ref, by_ref, baw_ref, bv_ref,
                 val_ref, ia_ref, ib_ref, ic_ref, id_ref,
                 wa_ref, wb_ref, wc_ref, wd_ref):
    i = pl.program_id(0)
    q = q_ref[...]

    val_ref[...] = lax.dot_general(
        inp_ref[...], wv_ref[...], (((1,), (1,)), ((), ())),
        preferred_element_type=jnp.float32) + bv_ref[...]

    sox = lax.dot_general(q, wx_ref[...], (((1,), (1,)), ((), ())),
                          preferred_element_type=jnp.float32) + bx_ref[...]
    soy = lax.dot_general(q, wy_ref[...], (((1,), (1,)), ((), ())),
                          preferred_element_type=jnp.float32) + by_ref[...]

    logit = lax.dot_general(q, waw_ref[...], (((1,), (1,)), ((), ())),
                            preferred_element_type=jnp.float32) + baw_ref[...]
    m = jnp.max(logit, axis=1, keepdims=True)
    e = jnp.exp(logit - m)
    si = lax.broadcasted_iota(jnp.int32, (LANES, LANES), 0)
    sj = lax.broadcasted_iota(jnp.int32, (LANES, LANES), 1)
    seg = ((si >> 4) == (sj >> 4)).astype(jnp.float32)
    denom = lax.dot_general(e, seg, (((1,), (0,)), ((), ())),
                            preferred_element_type=jnp.float32,
                            precision=lax.Precision.HIGHEST)
    aw = e / denom

    li = lax.broadcasted_iota(jnp.int32, (NL, LANES), 0)
    lj = lax.broadcasted_iota(jnp.int32, (NL, LANES), 1)
    exp_mat = (((lj >> 2) & 3) == li).astype(jnp.float32)
    refx = lax.dot_general(refx_ref[...], exp_mat, (((1,), (0,)), ((), ())),
                           preferred_element_type=jnp.float32,
                           precision=lax.Precision.HIGHEST)
    refy = lax.dot_general(refy_ref[...], exp_mat, (((1,), (0,)), ((), ())),
                           preferred_element_type=jnp.float32,
                           precision=lax.Precision.HIGHEST)

    lane = lax.broadcasted_iota(jnp.int32, (SBLK, LANES), 1)
    lvl = (lane >> 2) & 3
    h_lane = lane >> 4

    wi = jnp.full((SBLK, LANES), SPATIAL[0][1], jnp.int32)
    hi = jnp.full((SBLK, LANES), SPATIAL[0][0], jnp.int32)
    st = jnp.full((SBLK, LANES), STARTS[0], jnp.int32)
    for l in range(1, NL):
        wi = jnp.where(lvl == l, SPATIAL[l][1], wi)
        hi = jnp.where(lvl == l, SPATIAL[l][0], hi)
        st = jnp.where(lvl == l, STARTS[l], st)
    wf = wi.astype(jnp.float32)
    hf = hi.astype(jnp.float32)

    x = jnp.clip(refx + sox, 0.0, 1.0) * wf - 0.5
    y = jnp.clip(refy + soy, 0.0, 1.0) * hf - 0.5

    flx = jnp.floor(x).astype(jnp.int32)
    fly = jnp.floor(y).astype(jnp.int32)
    x0 = jnp.clip(flx, 0, wi - 1)
    x1 = jnp.clip(flx + 1, 0, wi - 1)
    y0 = jnp.clip(fly, 0, hi - 1)
    y1 = jnp.clip(fly + 1, 0, hi - 1)
    x0f = x0.astype(jnp.float32)
    x1f = x1.astype(jnp.float32)
    y0f = y0.astype(jnp.float32)
    y1f = y1.astype(jnp.float32)

    wa_ref[...] = aw * ((x1f - x) * (y1f - y))
    wb_ref[...] = aw * ((x1f - x) * (y - y0f))
    wc_ref[...] = aw * ((x - x0f) * (y1f - y))
    wd_ref[...] = aw * ((x - x0f) * (y - y0f))

    row0 = i * SBLK
    ridx = row0 + lax.broadcasted_iota(jnp.int32, (SBLK, LANES), 0)
    base = (ridx // LQ) * (LEN_IN * NH)

    def flat(yc, xc):
        return base + (st + yc * wi + xc) * NH + h_lane

    ia_ref[...] = flat(y0, x0)
    ib_ref[...] = flat(y1, x0)
    ic_ref[...] = flat(y0, x1)
    id_ref[...] = flat(y1, x1)


def _out_kernel(x_ref, wo_ref, bo_ref, o_ref):
    o_ref[...] = lax.dot_general(
        x_ref[...], wo_ref[...], (((1,), (1,)), ((), ())),
        preferred_element_type=jnp.float32) + bo_ref[...]


def _make_sc_sample():
    mesh = plsc.VectorSubcoreMesh(core_axis_name="c", subcore_axis_name="s")

    @functools.partial(
        pl.kernel, mesh=mesh,
        compiler_params=pltpu.CompilerParams(use_tc_tiling_on_sc=False),
        out_type=jax.ShapeDtypeStruct((ROWS, D), jnp.float32),
        scratch_types=[
            pltpu.VMEM((CH, LANES), jnp.int32),
            pltpu.VMEM((CH, LANES), jnp.int32),
            pltpu.VMEM((CH, LANES), jnp.int32),
            pltpu.VMEM((CH, LANES), jnp.int32),
            pltpu.VMEM((CH, LANES), jnp.float32),
            pltpu.VMEM((CH, LANES), jnp.float32),
            pltpu.VMEM((CH, LANES), jnp.float32),
            pltpu.VMEM((CH, LANES), jnp.float32),
            pltpu.VMEM((CH, LANES, HD), jnp.float32),
            pltpu.VMEM((CH, LANES, HD), jnp.float32),
            pltpu.VMEM((CH, LANES, HD), jnp.float32),
            pltpu.VMEM((CH, LANES, HD), jnp.float32),
            pltpu.VMEM((CH, D), jnp.float32),
            pltpu.SemaphoreType.DMA,
        ],
    )
    def sample(tab, ia, ib, ic, idd, wa, wb, wc, wd, out,
               iva, ivb, ivc, ivd, wva, wvb, wvc, wvd,
               ra, rb, rc, rd, ov, sem):
        wid = lax.axis_index("s") * 2 + lax.axis_index("c")
        base = wid * RW

        def step(g, carry):
            r0 = base + g * CH
            pltpu.sync_copy(ia.at[pl.ds(r0, CH)], iva)
            pltpu.sync_copy(ib.at[pl.ds(r0, CH)], ivb)
            pltpu.sync_copy(ic.at[pl.ds(r0, CH)], ivc)
            pltpu.sync_copy(idd.at[pl.ds(r0, CH)], ivd)
            pltpu.sync_copy(wa.at[pl.ds(r0, CH)], wva)
            pltpu.sync_copy(wb.at[pl.ds(r0, CH)], wvb)
            pltpu.sync_copy(wc.at[pl.ds(r0, CH)], wvc)
            pltpu.sync_copy(wd.at[pl.ds(r0, CH)], wvd)

            cps = []
            for iv, rv in ((iva, ra), (ivb, rb), (ivc, rc), (ivd, rd)):
                for k in range(CH):
                    cps.append(pltpu.async_copy(tab.at[iv.at[k]], rv.at[k], sem))
            for cp in cps:
                cp.wait()

            def kh_body(kh, carry2):
                k = kh // NH
                h = kh - k * NH
                hbase = h * (NL * NP)
                acc0 = jnp.zeros((16,), jnp.float32)
                acc1 = jnp.zeros((16,), jnp.float32)
                for wv_, rv_ in ((wva, ra), (wvb, rb), (wvc, rc), (wvd, rd)):
                    wrow = wv_[k, pl.ds(hbase, NL * NP)]
                    for lp in range(NL * NP):
                        wvec = jnp.broadcast_to(wrow[lp], (16,))
                        v0 = rv_[k, hbase + lp, pl.ds(0, 16)]
                        v1 = rv_[k, hbase + lp, pl.ds(16, 16)]
                        acc0 = acc0 + wvec * v0
                        acc1 = acc1 + wvec * v1
                ov[k, pl.ds(h * HD, 16)] = acc0
                ov[k, pl.ds(h * HD + 16, 16)] = acc1
                return carry2

            lax.fori_loop(0, CH * NH, kh_body, 0)
            pltpu.sync_copy(ov, out.at[pl.ds(r0, CH)])
            return carry

        lax.fori_loop(0, RW // CH, step, 0)

    return sample


_sc_cache = []


def _get_sc_sample():
    if not _sc_cache:
        _sc_cache.append(_make_sc_sample())
    return _sc_cache[0]


def kernel(query, reference_points, input_flatten, input_spatial_shapes,
           input_level_start_index, W_so, b_so, W_aw, b_aw, W_v, b_v,
           W_o, b_o):
    q2 = query.reshape(ROWS, D)
    inp2 = input_flatten.reshape(ROWS, D)
    refx = reference_points[..., 0].reshape(ROWS, NL)
    refy = reference_points[..., 1].reshape(ROWS, NL)
    Wx = W_so[0::2]
    Wy = W_so[1::2]
    bx = b_so[0::2].reshape(1, LANES)
    by = b_so[1::2].reshape(1, LANES)
    baw = b_aw.reshape(1, LANES)
    bv = b_v.reshape(1, D)
    bo = b_o.reshape(1, D)

    row_spec = pl.BlockSpec((SBLK, D), lambda i: (i, 0))
    lane_spec = pl.BlockSpec((SBLK, LANES), lambda i: (i, 0))
    ref_spec = pl.BlockSpec((SBLK, NL), lambda i: (i, 0))

    def full(shape):
        return pl.BlockSpec(shape, lambda i: tuple(0 for _ in shape))

    outs = pl.pallas_call(
        _proj_kernel,
        grid=(GRID,),
        in_specs=[
            row_spec, row_spec, ref_spec, ref_spec,
            full((LANES, D)), full((LANES, D)), full((LANES, D)),
            full((D, D)),
            full((1, LANES)), full((1, LANES)), full((1, LANES)),
            full((1, D)),
        ],
        out_specs=[
            row_spec,
            lane_spec, lane_spec, lane_spec, lane_spec,
            lane_spec, lane_spec, lane_spec, lane_spec,
        ],
        out_shape=[
            jax.ShapeDtypeStruct((ROWS, D), jnp.float32),
            jax.ShapeDtypeStruct((ROWS, LANES), jnp.int32),
            jax.ShapeDtypeStruct((ROWS, LANES), jnp.int32),
            jax.ShapeDtypeStruct((ROWS, LANES), jnp.int32),
            jax.ShapeDtypeStruct((ROWS, LANES), jnp.int32),
            jax.ShapeDtypeStruct((ROWS, LANES), jnp.float32),
            jax.ShapeDtypeStruct((ROWS, LANES), jnp.float32),
            jax.ShapeDtypeStruct((ROWS, LANES), jnp.float32),
            jax.ShapeDtypeStruct((ROWS, LANES), jnp.float32),
        ],
    )(q2, inp2, refx, refy, Wx, Wy, W_aw, W_v, bx, by, baw, bv)

    val, ia, ib, ic, idd, wa, wb, wc, wd = outs
    tab = val.reshape(ROWS * NH, HD)

    sampled = _get_sc_sample()(tab, ia, ib, ic, idd, wa, wb, wc, wd)

    out = pl.pallas_call(
        _out_kernel,
        grid=(GRID,),
        in_specs=[row_spec, full((D, D)), full((1, D))],
        out_specs=row_spec,
        out_shape=jax.ShapeDtypeStruct((ROWS, D), jnp.float32),
    )(sampled, W_o, bo)

    return out.reshape(B, LQ, D)
